# baseline (speedup 1.0000x reference)
import dataclasses
import functools
import math

import jax
import jax.numpy as jnp
from jax import lax
from jax.experimental import pallas as pl
from jax.experimental.pallas import tpu as pltpu
from jax.experimental.pallas import tpu_sc as plsc

N = 100000
IN_FEATURES = 128
HIDDEN = 64
NUM_SEGMENTS = 512
SHIFT = float(math.log(2.0))

BLOCK = 2048
N_PAD = 100352

NUM_SUBCORES = 16
LANES = 16
CHUNK = N_PAD // NUM_SUBCORES
ROWS_PER_SUBCORE = NUM_SEGMENTS // NUM_SUBCORES


PACK = 2


def _mlp_body(scaler_ref, w1_ref, b1_ref, w2_ref, b2_ref, q_ref):
    x = scaler_ref[...].astype(jnp.bfloat16)
    w1 = w1_ref[...].astype(jnp.bfloat16)
    h = jnp.dot(x, w1, preferred_element_type=jnp.float32) + b1_ref[...]
    h = jax.nn.softplus(h) - SHIFT
    q_ref[...] = jnp.dot(h, w2_ref[...], preferred_element_type=jnp.float32) + b2_ref[...]


def _mlp_call(scaler, W1, b1, W2, b2):
    rows_p = N_PAD // PACK
    block_p = BLOCK // PACK
    in_p = PACK * IN_FEATURES
    hid_p = PACK * HIDDEN
    z = jnp.zeros((IN_FEATURES, HIDDEN), jnp.float32)
    w1s = jnp.concatenate(
        [
            jnp.concatenate([W1, z], axis=1),
            jnp.concatenate([z, W1], axis=1),
        ],
        axis=0,
    )
    b1s = jnp.concatenate([b1, b1]).reshape(1, hid_p)
    z2 = jnp.zeros((HIDDEN, 1), jnp.float32)
    w2s = jnp.concatenate(
        [
            jnp.concatenate([W2, z2], axis=1),
            jnp.concatenate([z2, W2], axis=1),
        ],
        axis=0,
    )
    b2s = jnp.broadcast_to(b2.reshape(1, 1), (1, PACK))
    scaler_p = scaler.reshape(N // PACK, in_p)
    q = pl.pallas_call(
        _mlp_body,
        grid=(rows_p // block_p,),
        in_specs=[
            pl.BlockSpec((block_p, in_p), lambda i: (i, 0)),
            pl.BlockSpec((in_p, hid_p), lambda i: (0, 0)),
            pl.BlockSpec((1, hid_p), lambda i: (0, 0)),
            pl.BlockSpec((hid_p, PACK), lambda i: (0, 0)),
            pl.BlockSpec((1, PACK), lambda i: (0, 0)),
        ],
        out_specs=pl.BlockSpec((block_p, PACK), lambda i: (i, 0)),
        out_shape=jax.ShapeDtypeStruct((rows_p, PACK), jnp.float32),
    )(scaler_p, w1s, b1s, w2s, b2s)
    return q.reshape(N_PAD)


_SC_COMPILER_PARAMS = pltpu.CompilerParams()
if "needs_layout_passes" in pltpu.CompilerParams.__dataclass_fields__:
    _SC_COMPILER_PARAMS = dataclasses.replace(
        _SC_COMPILER_PARAMS, needs_layout_passes=False
    )


@functools.partial(
    pl.kernel,
    compiler_params=_SC_COMPILER_PARAMS,
    out_type=jax.ShapeDtypeStruct((NUM_SEGMENTS,), jnp.float32),
    mesh=plsc.VectorSubcoreMesh(core_axis_name="c", subcore_axis_name="s"),
    scratch_types=[
        pltpu.VMEM((CHUNK,), jnp.float32),
        pltpu.VMEM((CHUNK,), jnp.float32),
        pltpu.VMEM((CHUNK,), jnp.float32),
        pltpu.VMEM((CHUNK,), jnp.float32),
        pltpu.VMEM((CHUNK,), jnp.int32),
        pltpu.VMEM((NUM_SEGMENTS, LANES), jnp.float32),
        pltpu.VMEM((NUM_SEGMENTS,), jnp.float32),
        pltpu.VMEM((NUM_SUBCORES, NUM_SEGMENTS), jnp.float32),
        pltpu.VMEM((ROWS_PER_SUBCORE,), jnp.float32),
        pltpu.VMEM_SHARED((NUM_SUBCORES, NUM_SEGMENTS), jnp.float32),
    ],
)
def _segsum(
    q_hbm, pos_hbm, id_hbm, out_hbm,
    q_loc, x_loc, y_loc, z_loc, id_loc, acc, red, allred, outbuf, shared,
):
    cid = lax.axis_index("c")
    sid = lax.axis_index("s")

    @pl.when(cid == 0)
    def _():
        base = sid * CHUNK
        pltpu.sync_copy(q_hbm.at[pl.ds(base, CHUNK)], q_loc)
        pltpu.sync_copy(pos_hbm.at[pl.ds(base, CHUNK)], x_loc)
        pltpu.sync_copy(pos_hbm.at[pl.ds(N_PAD + base, CHUNK)], y_loc)
        pltpu.sync_copy(pos_hbm.at[pl.ds(2 * N_PAD + base, CHUNK)], z_loc)
        pltpu.sync_copy(id_hbm.at[pl.ds(base, CHUNK)], id_loc)

        zeros16 = jnp.zeros((LANES,), jnp.float32)
        lane = lax.iota(jnp.int32, LANES)

        @pl.loop(0, NUM_SEGMENTS)
        def _(r):
            acc[r] = zeros16

        @pl.loop(0, CHUNK, step=LANES)
        def _(i):
            ids = id_loc[pl.ds(i, LANES)]
            xs = x_loc[pl.ds(i, LANES)]
            ys = y_loc[pl.ds(i, LANES)]
            zs = z_loc[pl.ds(i, LANES)]
            r2 = xs * xs + ys * ys + zs * zs
            vals = jnp.where(r2 > 0.0, q_loc[pl.ds(i, LANES)] * r2, 0.0)
            plsc.addupdate_scatter(acc, [ids, lane], vals)

        @pl.loop(0, NUM_SEGMENTS, step=LANES)
        def _(r):
            rows = r + lane
            tot = zeros16
            for l in range(LANES):
                col = jnp.full((LANES,), l, jnp.int32)
                tot = tot + plsc.load_gather(acc, [rows, col])
            red[pl.ds(r, LANES)] = tot

        pltpu.sync_copy(red, shared.at[sid])
        plsc.subcore_barrier()
        pltpu.sync_copy(shared, allred)

        rowbase = sid * ROWS_PER_SUBCORE
        for rc in range(ROWS_PER_SUBCORE // LANES):
            tot = zeros16
            for s in range(NUM_SUBCORES):
                tot = tot + allred[s, pl.ds(rowbase + rc * LANES, LANES)]
            outbuf[pl.ds(rc * LANES, LANES)] = tot
        pltpu.sync_copy(outbuf, out_hbm.at[pl.ds(rowbase, ROWS_PER_SUBCORE)])


def kernel(pos, scaler, vector, W1, b1, W2, b2, batch_index):
    del vector
    ids = jnp.pad(batch_index.astype(jnp.int32), (0, N_PAD - N))
    pos_t = jnp.pad(pos.T, ((0, 0), (0, N_PAD - N))).reshape(3 * N_PAD)
    q = _mlp_call(scaler, W1, b1, W2, b2)
    out = _segsum(q, pos_t, ids)
    return out.reshape(NUM_SEGMENTS, 1)


# speedup vs baseline: 1.3487x; 1.3487x over previous
import dataclasses
import functools
import math

import jax
import jax.numpy as jnp
from jax import lax
from jax.experimental import pallas as pl
from jax.experimental.pallas import tpu as pltpu
from jax.experimental.pallas import tpu_sc as plsc

N = 100000
IN_FEATURES = 128
HIDDEN = 64
NUM_SEGMENTS = 512
SHIFT = float(math.log(2.0))

BLOCK = 2048
N_PAD = 100352

NUM_SUBCORES = 16
LANES = 16
CHUNK = N_PAD // NUM_SUBCORES
ROWS_PER_SUBCORE = NUM_SEGMENTS // NUM_SUBCORES


def _mlp_body(scaler_ref, w1_ref, b1_ref, w2_ref, b2_ref, q_ref):
    x = scaler_ref[...].astype(jnp.bfloat16)
    w1 = w1_ref[...].astype(jnp.bfloat16)
    h = jnp.dot(x, w1, preferred_element_type=jnp.float32) + b1_ref[...]
    h = jax.nn.softplus(h) - SHIFT
    q_ref[...] = jnp.dot(h, w2_ref[...], preferred_element_type=jnp.float32) + b2_ref[...]


def _mlp_call(scaler, W1, b1, W2, b2):
    q = pl.pallas_call(
        _mlp_body,
        grid=(N_PAD // BLOCK,),
        in_specs=[
            pl.BlockSpec((BLOCK, IN_FEATURES), lambda i: (i, 0)),
            pl.BlockSpec((IN_FEATURES, HIDDEN), lambda i: (0, 0)),
            pl.BlockSpec((1, HIDDEN), lambda i: (0, 0)),
            pl.BlockSpec((HIDDEN, 1), lambda i: (0, 0)),
            pl.BlockSpec((1, 1), lambda i: (0, 0)),
        ],
        out_specs=pl.BlockSpec((BLOCK, 1), lambda i: (i, 0)),
        out_shape=jax.ShapeDtypeStruct((N_PAD, 1), jnp.float32),
    )(scaler, W1, b1.reshape(1, HIDDEN), W2, b2.reshape(1, 1))
    return q.reshape(N_PAD)


_SC_COMPILER_PARAMS = pltpu.CompilerParams()
if "needs_layout_passes" in pltpu.CompilerParams.__dataclass_fields__:
    _SC_COMPILER_PARAMS = dataclasses.replace(
        _SC_COMPILER_PARAMS, needs_layout_passes=False
    )


@functools.partial(
    pl.kernel,
    compiler_params=_SC_COMPILER_PARAMS,
    out_type=jax.ShapeDtypeStruct((NUM_SEGMENTS,), jnp.float32),
    mesh=plsc.VectorSubcoreMesh(core_axis_name="c", subcore_axis_name="s"),
    scratch_types=[
        pltpu.VMEM((CHUNK,), jnp.float32),
        pltpu.VMEM((CHUNK,), jnp.float32),
        pltpu.VMEM((CHUNK,), jnp.float32),
        pltpu.VMEM((CHUNK,), jnp.float32),
        pltpu.VMEM((CHUNK,), jnp.int32),
        pltpu.VMEM((NUM_SEGMENTS, LANES), jnp.float32),
        pltpu.VMEM((NUM_SEGMENTS,), jnp.float32),
        pltpu.VMEM((NUM_SUBCORES, NUM_SEGMENTS), jnp.float32),
        pltpu.VMEM((ROWS_PER_SUBCORE,), jnp.float32),
        pltpu.VMEM_SHARED((NUM_SUBCORES, NUM_SEGMENTS), jnp.float32),
    ],
)
def _segsum(
    q_hbm, pos_hbm, id_hbm, out_hbm,
    q_loc, x_loc, y_loc, z_loc, id_loc, acc, red, allred, outbuf, shared,
):
    cid = lax.axis_index("c")
    sid = lax.axis_index("s")

    @pl.when(cid == 0)
    def _():
        base = sid * CHUNK
        pltpu.sync_copy(q_hbm.at[pl.ds(base, CHUNK)], q_loc)
        pltpu.sync_copy(pos_hbm.at[pl.ds(base, CHUNK)], x_loc)
        pltpu.sync_copy(pos_hbm.at[pl.ds(N_PAD + base, CHUNK)], y_loc)
        pltpu.sync_copy(pos_hbm.at[pl.ds(2 * N_PAD + base, CHUNK)], z_loc)
        pltpu.sync_copy(id_hbm.at[pl.ds(base, CHUNK)], id_loc)

        zeros16 = jnp.zeros((LANES,), jnp.float32)
        lane = lax.iota(jnp.int32, LANES)

        @pl.loop(0, NUM_SEGMENTS)
        def _(r):
            acc[r] = zeros16

        @pl.loop(0, CHUNK, step=LANES)
        def _(i):
            ids = id_loc[pl.ds(i, LANES)]
            xs = x_loc[pl.ds(i, LANES)]
            ys = y_loc[pl.ds(i, LANES)]
            zs = z_loc[pl.ds(i, LANES)]
            r2 = xs * xs + ys * ys + zs * zs
            vals = jnp.where(r2 > 0.0, q_loc[pl.ds(i, LANES)] * r2, 0.0)
            plsc.addupdate_scatter(acc, [ids, lane], vals)

        @pl.loop(0, NUM_SEGMENTS, step=LANES)
        def _(r):
            rows = r + lane
            tot = zeros16
            for l in range(LANES):
                col = jnp.full((LANES,), l, jnp.int32)
                tot = tot + plsc.load_gather(acc, [rows, col])
            red[pl.ds(r, LANES)] = tot

        pltpu.sync_copy(red, shared.at[sid])
        plsc.subcore_barrier()
        pltpu.sync_copy(shared, allred)

        rowbase = sid * ROWS_PER_SUBCORE
        for rc in range(ROWS_PER_SUBCORE // LANES):
            tot = zeros16
            for s in range(NUM_SUBCORES):
                tot = tot + allred[s, pl.ds(rowbase + rc * LANES, LANES)]
            outbuf[pl.ds(rc * LANES, LANES)] = tot
        pltpu.sync_copy(outbuf, out_hbm.at[pl.ds(rowbase, ROWS_PER_SUBCORE)])


def kernel(pos, scaler, vector, W1, b1, W2, b2, batch_index):
    del vector
    ids = jnp.pad(batch_index.astype(jnp.int32), (0, N_PAD - N))
    pos_t = jnp.pad(pos.T, ((0, 0), (0, N_PAD - N))).reshape(3 * N_PAD)
    q = _mlp_call(scaler, W1, b1, W2, b2)
    out = _segsum(q, pos_t, ids)
    return out.reshape(NUM_SEGMENTS, 1)


# speedup vs baseline: 1.5839x; 1.1744x over previous
import dataclasses
import functools
import math

import jax
import jax.numpy as jnp
from jax import lax
from jax.experimental import pallas as pl
from jax.experimental.pallas import tpu as pltpu
from jax.experimental.pallas import tpu_sc as plsc

N = 100000
IN_FEATURES = 128
HIDDEN = 64
NUM_SEGMENTS = 512
SHIFT = float(math.log(2.0))

BLOCK = 8192
N_PAD = 106496

NUM_SUBCORES = 16
LANES = 16
CHUNK = N_PAD // NUM_SUBCORES
ROWS_PER_SUBCORE = NUM_SEGMENTS // NUM_SUBCORES


def _mlp_body(scaler_ref, w1_ref, b1_ref, w2_ref, b2_ref, q_ref):
    x = scaler_ref[...].astype(jnp.bfloat16)
    w1 = w1_ref[...].astype(jnp.bfloat16)
    h = jnp.dot(x, w1, preferred_element_type=jnp.float32) + b1_ref[...]
    h = jax.nn.softplus(h) - SHIFT
    q_ref[...] = jnp.dot(h, w2_ref[...], preferred_element_type=jnp.float32) + b2_ref[...]


def _mlp_call(scaler, W1, b1, W2, b2):
    q = pl.pallas_call(
        _mlp_body,
        grid=(N_PAD // BLOCK,),
        in_specs=[
            pl.BlockSpec((BLOCK, IN_FEATURES), lambda i: (i, 0)),
            pl.BlockSpec((IN_FEATURES, HIDDEN), lambda i: (0, 0)),
            pl.BlockSpec((1, HIDDEN), lambda i: (0, 0)),
            pl.BlockSpec((HIDDEN, 1), lambda i: (0, 0)),
            pl.BlockSpec((1, 1), lambda i: (0, 0)),
        ],
        out_specs=pl.BlockSpec((BLOCK, 1), lambda i: (i, 0)),
        out_shape=jax.ShapeDtypeStruct((N_PAD, 1), jnp.float32),
    )(scaler, W1, b1.reshape(1, HIDDEN), W2, b2.reshape(1, 1))
    return q.reshape(N_PAD)


_SC_COMPILER_PARAMS = pltpu.CompilerParams()
if "needs_layout_passes" in pltpu.CompilerParams.__dataclass_fields__:
    _SC_COMPILER_PARAMS = dataclasses.replace(
        _SC_COMPILER_PARAMS, needs_layout_passes=False
    )


@functools.partial(
    pl.kernel,
    compiler_params=_SC_COMPILER_PARAMS,
    out_type=jax.ShapeDtypeStruct((NUM_SEGMENTS,), jnp.float32),
    mesh=plsc.VectorSubcoreMesh(core_axis_name="c", subcore_axis_name="s"),
    scratch_types=[
        pltpu.VMEM((CHUNK,), jnp.float32),
        pltpu.VMEM((CHUNK,), jnp.float32),
        pltpu.VMEM((CHUNK,), jnp.float32),
        pltpu.VMEM((CHUNK,), jnp.float32),
        pltpu.VMEM((CHUNK,), jnp.int32),
        pltpu.VMEM((NUM_SEGMENTS, LANES), jnp.float32),
        pltpu.VMEM((NUM_SEGMENTS,), jnp.float32),
        pltpu.VMEM((NUM_SUBCORES, NUM_SEGMENTS), jnp.float32),
        pltpu.VMEM((ROWS_PER_SUBCORE,), jnp.float32),
        pltpu.VMEM_SHARED((NUM_SUBCORES, NUM_SEGMENTS), jnp.float32),
    ],
)
def _segsum(
    q_hbm, pos_hbm, id_hbm, out_hbm,
    q_loc, x_loc, y_loc, z_loc, id_loc, acc, red, allred, outbuf, shared,
):
    cid = lax.axis_index("c")
    sid = lax.axis_index("s")

    @pl.when(cid == 0)
    def _():
        base = sid * CHUNK
        pltpu.sync_copy(q_hbm.at[pl.ds(base, CHUNK)], q_loc)
        pltpu.sync_copy(pos_hbm.at[pl.ds(base, CHUNK)], x_loc)
        pltpu.sync_copy(pos_hbm.at[pl.ds(N_PAD + base, CHUNK)], y_loc)
        pltpu.sync_copy(pos_hbm.at[pl.ds(2 * N_PAD + base, CHUNK)], z_loc)
        pltpu.sync_copy(id_hbm.at[pl.ds(base, CHUNK)], id_loc)

        zeros16 = jnp.zeros((LANES,), jnp.float32)
        lane = lax.iota(jnp.int32, LANES)

        @pl.loop(0, NUM_SEGMENTS)
        def _(r):
            acc[r] = zeros16

        @pl.loop(0, CHUNK, step=LANES)
        def _(i):
            ids = id_loc[pl.ds(i, LANES)]
            xs = x_loc[pl.ds(i, LANES)]
            ys = y_loc[pl.ds(i, LANES)]
            zs = z_loc[pl.ds(i, LANES)]
            r2 = xs * xs + ys * ys + zs * zs
            vals = jnp.where(r2 > 0.0, q_loc[pl.ds(i, LANES)] * r2, 0.0)
            plsc.addupdate_scatter(acc, [ids, lane], vals)

        @pl.loop(0, NUM_SEGMENTS, step=LANES)
        def _(r):
            rows = r + lane
            tot = zeros16
            for l in range(LANES):
                col = jnp.full((LANES,), l, jnp.int32)
                tot = tot + plsc.load_gather(acc, [rows, col])
            red[pl.ds(r, LANES)] = tot

        pltpu.sync_copy(red, shared.at[sid])
        plsc.subcore_barrier()
        pltpu.sync_copy(shared, allred)

        rowbase = sid * ROWS_PER_SUBCORE
        for rc in range(ROWS_PER_SUBCORE // LANES):
            tot = zeros16
            for s in range(NUM_SUBCORES):
                tot = tot + allred[s, pl.ds(rowbase + rc * LANES, LANES)]
            outbuf[pl.ds(rc * LANES, LANES)] = tot
        pltpu.sync_copy(outbuf, out_hbm.at[pl.ds(rowbase, ROWS_PER_SUBCORE)])


def kernel(pos, scaler, vector, W1, b1, W2, b2, batch_index):
    del vector
    ids = jnp.pad(batch_index.astype(jnp.int32), (0, N_PAD - N))
    pos_t = jnp.pad(pos.T, ((0, 0), (0, N_PAD - N))).reshape(3 * N_PAD)
    q = _mlp_call(scaler, W1, b1, W2, b2)
    out = _segsum(q, pos_t, ids)
    return out.reshape(NUM_SEGMENTS, 1)
